# Initial kernel scaffold; baseline (speedup 1.0000x reference)
import jax
import jax.numpy as jnp
from jax.experimental import pallas as pl


def kernel(x_feat, edge_index, bases, W_pre, b_pre, W1, b1, g1, be1, W2, b2, g2, be2):
    raise NotImplementedError("write your pallas kernel here")


# speedup vs baseline: 3.2318x; 3.2318x over previous
import functools

import jax
import jax.numpy as jnp
from jax import lax
from jax.experimental import pallas as pl
from jax.experimental.pallas import tpu as pltpu
from jax.experimental.pallas import tpu_sc as plsc

N = 10000
E = 320000
D = 128

_NC = 2
_NS = 16
_L = 16
_NW = _NC * _NS
_EPW = E // _NW
_CH = 80
_NCHUNK = _EPW // _CH
_RPT = 624
_TAIL = N - _RPT * _NS


def _gelu(z):
    return 0.5 * z * (1.0 + lax.erf(z * (2.0 ** -0.5)))


def _sc_aggregate(h, x_feat, src, dst, bases):
    mesh = plsc.VectorSubcoreMesh(core_axis_name="c", subcore_axis_name="s")

    @functools.partial(
        pl.kernel,
        mesh=mesh,
        out_type=jax.ShapeDtypeStruct((_NC, N, D), jnp.float32),
        scratch_types=[
            pltpu.VMEM((_CH,), jnp.int32),
            pltpu.VMEM((_CH,), jnp.int32),
            pltpu.VMEM((_CH, D), jnp.float32),
            pltpu.VMEM((_CH, D), jnp.float32),
            pltpu.VMEM((_CH, D), jnp.float32),
            pltpu.VMEM_SHARED((N, D), jnp.float32),
            pltpu.SemaphoreType.DMA,
        ],
    )
    def k(h_hbm, x_hbm, src_hbm, dst_hbm, bases_hbm, out_hbm,
          srcv, dstv, hv, bv, mv, acc_sh, sem):
        c = lax.axis_index("c")
        s = lax.axis_index("s")
        wid = s * _NC + c
        r0 = s * _RPT
        pltpu.sync_copy(x_hbm.at[pl.ds(r0, _RPT)], acc_sh.at[pl.ds(r0, _RPT)])

        @pl.when(s == _NS - 1)
        def _():
            pltpu.sync_copy(x_hbm.at[pl.ds(_RPT * _NS, _TAIL)],
                            acc_sh.at[pl.ds(_RPT * _NS, _TAIL)])

        plsc.subcore_barrier()

        ebase = wid * _EPW

        def chunk(i, carry):
            base = ebase + i * _CH
            pltpu.sync_copy(src_hbm.at[pl.ds(base, _CH)], srcv)
            pltpu.sync_copy(dst_hbm.at[pl.ds(base, _CH)], dstv)
            pltpu.async_copy(h_hbm.at[srcv], hv, sem).wait()
            pltpu.sync_copy(bases_hbm.at[pl.ds(base, _CH)], bv)

            def mul(e, cc):
                for j in range(D // _L):
                    sl = pl.ds(j * _L, _L)
                    mv[e, sl] = hv[e, sl] * bv[e, sl]
                return cc

            lax.fori_loop(0, _CH, mul, 0)
            pltpu.sync_copy(mv, acc_sh.at[dstv], add=True)
            return carry

        lax.fori_loop(0, _NCHUNK, chunk, 0)
        plsc.subcore_barrier()
        pltpu.sync_copy(acc_sh.at[pl.ds(r0, _RPT)],
                        out_hbm.at[c, pl.ds(r0, _RPT)])

        @pl.when(s == _NS - 1)
        def _():
            pltpu.sync_copy(acc_sh.at[pl.ds(_RPT * _NS, _TAIL)],
                            out_hbm.at[c, pl.ds(_RPT * _NS, _TAIL)])

    return k(h, x_feat, src, dst, bases)


def _tc_preffn(x, W, b):
    def body(x_ref, w_ref, b_ref, o_ref):
        z = jnp.dot(x_ref[...], w_ref[...],
                    preferred_element_type=jnp.float32,
                    precision=lax.Precision.HIGHEST) + b_ref[...]
        o_ref[...] = _gelu(z)

    return pl.pallas_call(
        body,
        out_shape=jax.ShapeDtypeStruct((N, D), jnp.float32),
    )(x, W, b.reshape(1, D))


def _bn(z, g, b):
    mu = jnp.mean(z, axis=0, keepdims=True)
    var = jnp.mean((z - mu) ** 2, axis=0, keepdims=True)
    return (z - mu) / jnp.sqrt(var + 1e-5) * g + b


def _tc_ffn(x_feat, aggr, W1, b1, g1, be1, W2, b2, g2, be2):
    def body(xf, ag, w1, b1r, g1r, be1r, w2, b2r, g2r, be2r, o_ref):
        x = ag[0] + ag[1] - xf[...]
        y = jnp.dot(x, w1[...], preferred_element_type=jnp.float32,
                    precision=lax.Precision.HIGHEST) + b1r[...]
        y = _gelu(_bn(y, g1r[...], be1r[...]))
        y = jnp.dot(y, w2[...], preferred_element_type=jnp.float32,
                    precision=lax.Precision.HIGHEST) + b2r[...]
        y = _gelu(_bn(y, g2r[...], be2r[...]))
        o_ref[...] = x + y

    r = lambda v: v.reshape(1, D)
    return pl.pallas_call(
        body,
        out_shape=jax.ShapeDtypeStruct((N, D), jnp.float32),
    )(x_feat, aggr, W1, r(b1), r(g1), r(be1), W2, r(b2), r(g2), r(be2))


def kernel(x_feat, edge_index, bases, W_pre, b_pre, W1, b1, g1, be1, W2, b2, g2, be2):
    ei = edge_index.astype(jnp.int32)
    src = ei[0]
    dst = ei[1]
    h = _tc_preffn(x_feat, W_pre, b_pre)
    aggr = _sc_aggregate(h, x_feat, src, dst, bases)
    return _tc_ffn(x_feat, aggr, W1, b1, g1, be1, W2, b2, g2, be2)


# speedup vs baseline: 6.5658x; 2.0316x over previous
import functools

import jax
import jax.numpy as jnp
from jax import lax
from jax.experimental import pallas as pl
from jax.experimental.pallas import tpu as pltpu
from jax.experimental.pallas import tpu_sc as plsc

N = 10000
E = 320000
D = 128

_NC = 2
_NS = 16
_L = 16
_NW = _NC * _NS
_EPW = E // _NW
_CH = 40
_NCHUNK = _EPW // _CH
_RPT = 624
_TAIL = N - _RPT * _NS


def _gelu(z):
    return 0.5 * z * (1.0 + lax.erf(z * (2.0 ** -0.5)))


def _sc_aggregate(h, x_feat, src, dst, bases):
    mesh = plsc.VectorSubcoreMesh(core_axis_name="c", subcore_axis_name="s")

    @functools.partial(
        pl.kernel,
        mesh=mesh,
        out_type=jax.ShapeDtypeStruct((_NC, N, D), jnp.float32),
        scratch_types=[
            [pltpu.VMEM((1, _CH), jnp.int32) for _ in range(4)],
            [pltpu.VMEM((1, _CH), jnp.int32) for _ in range(4)],
            [pltpu.VMEM((_CH, D), jnp.float32) for _ in range(2)],
            [pltpu.VMEM((_CH, D), jnp.float32) for _ in range(2)],
            [pltpu.VMEM((_CH, D), jnp.float32) for _ in range(2)],
            pltpu.VMEM_SHARED((N, D), jnp.float32),
            [pltpu.SemaphoreType.DMA for _ in range(4)],
            [pltpu.SemaphoreType.DMA for _ in range(4)],
            [pltpu.SemaphoreType.DMA for _ in range(2)],
            [pltpu.SemaphoreType.DMA for _ in range(2)],
            [pltpu.SemaphoreType.DMA for _ in range(2)],
        ],
    )
    def k(h_hbm, x_hbm, src_hbm, dst_hbm, bases_hbm, out_hbm,
          sidx, didx, hv, bv, mv, acc_sh,
          sem_si, sem_di, sem_h, sem_b, sem_s):
        c = lax.axis_index("c")
        s = lax.axis_index("s")
        wid = s * _NC + c
        r0 = s * _RPT
        pltpu.sync_copy(x_hbm.at[pl.ds(r0, _RPT)], acc_sh.at[pl.ds(r0, _RPT)])

        @pl.when(s == _NS - 1)
        def _():
            pltpu.sync_copy(x_hbm.at[pl.ds(_RPT * _NS, _TAIL)],
                            acc_sh.at[pl.ds(_RPT * _NS, _TAIL)])

        plsc.subcore_barrier()

        ebase = wid * _EPW

        def start_sidx(i, q):
            pltpu.async_copy(src_hbm.at[wid, i], sidx[q], sem_si[q])

        def start_didx(i, q):
            pltpu.async_copy(dst_hbm.at[wid, i], didx[q], sem_di[q])

        def start_gather(i, b, q):
            pltpu.async_copy(h_hbm.at[sidx[q].at[0]], hv[b], sem_h[b])
            pltpu.async_copy(bases_hbm.at[pl.ds(ebase + i * _CH, _CH)],
                             bv[b], sem_b[b])

        for q in range(4):
            start_sidx(q, q)
        for q in range(2):
            start_didx(q, q)
        for b in range(2):
            pltpu.make_async_copy(src_hbm.at[wid, b], sidx[b], sem_si[b]).wait()
            start_gather(b, b, b)

        def chunk(i, b, q):
            q2 = (q + 2) % 4
            pltpu.make_async_copy(h_hbm.at[sidx[q].at[0]], hv[b],
                                  sem_h[b]).wait()
            pltpu.make_async_copy(
                bases_hbm.at[pl.ds(ebase + i * _CH, _CH)], bv[b],
                sem_b[b]).wait()

            @pl.when(i >= 2)
            def _():
                pltpu.make_async_copy(mv[b], acc_sh.at[didx[q].at[0]],
                                      sem_s[b]).wait()

            @pl.when(i + 2 < _NCHUNK)
            def _():
                start_didx(i + 2, q2)

            def mul(e, cc):
                for j in range(D // _L):
                    sl = pl.ds(j * _L, _L)
                    mv[b][e, sl] = hv[b][e, sl] * bv[b][e, sl]
                return cc

            lax.fori_loop(0, _CH, mul, 0)

            pltpu.make_async_copy(dst_hbm.at[wid, i], didx[q], sem_di[q]).wait()
            pltpu.async_copy(mv[b], acc_sh.at[didx[q].at[0]], sem_s[b], add=True)

            @pl.when(i + 4 < _NCHUNK)
            def _():
                start_sidx(i + 4, q)

            @pl.when(i + 2 < _NCHUNK)
            def _():
                pltpu.make_async_copy(src_hbm.at[wid, i + 2], sidx[q2],
                                      sem_si[q2]).wait()
                start_gather(i + 2, b, q2)

        def quad(i4, carry):
            for qq in range(4):
                chunk(i4 * 4 + qq, qq % 2, qq)
            return carry

        lax.fori_loop(0, _NCHUNK // 4, quad, 0)
        for i in range(_NCHUNK - _NCHUNK % 4, _NCHUNK):
            chunk(i, i % 2, i % 4)
        for b in range(2):
            pltpu.make_async_copy(
                mv[b], acc_sh.at[didx[(_NCHUNK - 2 + b) % 4].at[0]],
                sem_s[b]).wait()
        plsc.subcore_barrier()
        pltpu.sync_copy(acc_sh.at[pl.ds(r0, _RPT)],
                        out_hbm.at[c, pl.ds(r0, _RPT)])

        @pl.when(s == _NS - 1)
        def _():
            pltpu.sync_copy(acc_sh.at[pl.ds(_RPT * _NS, _TAIL)],
                            out_hbm.at[c, pl.ds(_RPT * _NS, _TAIL)])

    return k(h, x_feat, src, dst, bases)


def _tc_preffn(x, W, b):
    def body(x_ref, w_ref, b_ref, o_ref):
        z = jnp.dot(x_ref[...], w_ref[...],
                    preferred_element_type=jnp.float32,
                    precision=lax.Precision.HIGHEST) + b_ref[...]
        o_ref[...] = _gelu(z)

    return pl.pallas_call(
        body,
        out_shape=jax.ShapeDtypeStruct((N, D), jnp.float32),
    )(x, W, b.reshape(1, D))


def _bn(z, g, b):
    mu = jnp.mean(z, axis=0, keepdims=True)
    var = jnp.mean((z - mu) ** 2, axis=0, keepdims=True)
    return (z - mu) / jnp.sqrt(var + 1e-5) * g + b


def _tc_ffn(x_feat, aggr, W1, b1, g1, be1, W2, b2, g2, be2):
    def body(xf, ag, w1, b1r, g1r, be1r, w2, b2r, g2r, be2r, o_ref):
        x = ag[0] + ag[1] - xf[...]
        y = jnp.dot(x, w1[...], preferred_element_type=jnp.float32,
                    precision=lax.Precision.HIGHEST) + b1r[...]
        y = _gelu(_bn(y, g1r[...], be1r[...]))
        y = jnp.dot(y, w2[...], preferred_element_type=jnp.float32,
                    precision=lax.Precision.HIGHEST) + b2r[...]
        y = _gelu(_bn(y, g2r[...], be2r[...]))
        o_ref[...] = x + y

    r = lambda v: v.reshape(1, D)
    return pl.pallas_call(
        body,
        out_shape=jax.ShapeDtypeStruct((N, D), jnp.float32),
    )(x_feat, aggr, W1, r(b1), r(g1), r(be1), W2, r(b2), r(g2), r(be2))


def kernel(x_feat, edge_index, bases, W_pre, b_pre, W1, b1, g1, be1, W2, b2, g2, be2):
    ei = edge_index.astype(jnp.int32)
    src = ei[0].reshape(_NW, _NCHUNK, 1, _CH)
    dst = ei[1].reshape(_NW, _NCHUNK, 1, _CH)
    h = _tc_preffn(x_feat, W_pre, b_pre)
    aggr = _sc_aggregate(h, x_feat, src, dst, bases)
    return _tc_ffn(x_feat, aggr, W1, b1, g1, be1, W2, b2, g2, be2)


# speedup vs baseline: 7.2550x; 1.1050x over previous
import functools

import jax
import jax.numpy as jnp
from jax import lax
from jax.experimental import pallas as pl
from jax.experimental.pallas import tpu as pltpu
from jax.experimental.pallas import tpu_sc as plsc

N = 10000
E = 320000
D = 128

_NC = 2
_NS = 16
_L = 16
_NW = _NC * _NS
_EPW = E // _NW
_CH = 40
_NCHUNK = _EPW // _CH
_RPT = 624
_TAIL = N - _RPT * _NS


def _gelu(z):
    return 0.5 * z * (1.0 + lax.erf(z * (2.0 ** -0.5)))


def _sc_aggregate(h, x_feat, src, dst, bases):
    mesh = plsc.VectorSubcoreMesh(core_axis_name="c", subcore_axis_name="s")

    @functools.partial(
        pl.kernel,
        mesh=mesh,
        out_type=jax.ShapeDtypeStruct((_NC, N, D), jnp.float32),
        scratch_types=[
            [pltpu.VMEM((1, _CH), jnp.int32) for _ in range(6)],
            [pltpu.VMEM((1, _CH), jnp.int32) for _ in range(6)],
            [pltpu.VMEM((_CH, D), jnp.float32) for _ in range(3)],
            [pltpu.VMEM((_CH, D), jnp.float32) for _ in range(3)],
            [pltpu.VMEM((_CH, D), jnp.float32) for _ in range(3)],
            pltpu.VMEM_SHARED((N, D), jnp.float32),
            [pltpu.SemaphoreType.DMA for _ in range(6)],
            [pltpu.SemaphoreType.DMA for _ in range(6)],
            [pltpu.SemaphoreType.DMA for _ in range(3)],
            [pltpu.SemaphoreType.DMA for _ in range(3)],
            [pltpu.SemaphoreType.DMA for _ in range(3)],
        ],
    )
    def k(h_hbm, x_hbm, src_hbm, dst_hbm, bases_hbm, out_hbm,
          sidx, didx, hv, bv, mv, acc_sh,
          sem_si, sem_di, sem_h, sem_b, sem_s):
        c = lax.axis_index("c")
        s = lax.axis_index("s")
        wid = s * _NC + c
        r0 = s * _RPT
        pltpu.sync_copy(x_hbm.at[pl.ds(r0, _RPT)], acc_sh.at[pl.ds(r0, _RPT)])

        @pl.when(s == _NS - 1)
        def _():
            pltpu.sync_copy(x_hbm.at[pl.ds(_RPT * _NS, _TAIL)],
                            acc_sh.at[pl.ds(_RPT * _NS, _TAIL)])

        plsc.subcore_barrier()

        ebase = wid * _EPW

        def start_sidx(i, q):
            pltpu.async_copy(src_hbm.at[wid, i], sidx[q], sem_si[q])

        def start_didx(i, q):
            pltpu.async_copy(dst_hbm.at[wid, i], didx[q], sem_di[q])

        def start_gather(i, b, q):
            pltpu.async_copy(h_hbm.at[sidx[q].at[0]], hv[b], sem_h[b])
            pltpu.async_copy(bases_hbm.at[pl.ds(ebase + i * _CH, _CH)],
                             bv[b], sem_b[b])

        for q in range(6):
            start_sidx(q, q)
        for q in range(3):
            start_didx(q, q)
        for b in range(3):
            pltpu.make_async_copy(src_hbm.at[wid, b], sidx[b], sem_si[b]).wait()
            start_gather(b, b, b)

        def chunk(i, b, q):
            q3 = (q + 3) % 6
            pltpu.make_async_copy(h_hbm.at[sidx[q].at[0]], hv[b],
                                  sem_h[b]).wait()
            pltpu.make_async_copy(
                bases_hbm.at[pl.ds(ebase + i * _CH, _CH)], bv[b],
                sem_b[b]).wait()

            @pl.when(i >= 3)
            def _():
                pltpu.make_async_copy(mv[b], acc_sh.at[didx[q].at[0]],
                                      sem_s[b]).wait()

            @pl.when(i + 3 < _NCHUNK)
            def _():
                start_didx(i + 3, q3)

            @plsc.parallel_loop(0, _CH, 1, unroll=4)
            def _mul(e):
                for j in range(D // _L):
                    sl = pl.ds(j * _L, _L)
                    mv[b][e, sl] = hv[b][e, sl] * bv[b][e, sl]

            pltpu.make_async_copy(dst_hbm.at[wid, i], didx[q], sem_di[q]).wait()
            pltpu.async_copy(mv[b], acc_sh.at[didx[q].at[0]], sem_s[b], add=True)

            @pl.when(i + 6 < _NCHUNK)
            def _():
                start_sidx(i + 6, q)

            @pl.when(i + 3 < _NCHUNK)
            def _():
                pltpu.make_async_copy(src_hbm.at[wid, i + 3], sidx[q3],
                                      sem_si[q3]).wait()
                start_gather(i + 3, b, q3)

        def six(i6, carry):
            for kk in range(6):
                chunk(i6 * 6 + kk, kk % 3, kk)
            return carry

        lax.fori_loop(0, _NCHUNK // 6, six, 0)
        for i in range(_NCHUNK - _NCHUNK % 6, _NCHUNK):
            chunk(i, i % 3, i % 6)
        for i in range(_NCHUNK - 3, _NCHUNK):
            pltpu.make_async_copy(
                mv[i % 3], acc_sh.at[didx[i % 6].at[0]],
                sem_s[i % 3]).wait()
        plsc.subcore_barrier()
        pltpu.sync_copy(acc_sh.at[pl.ds(r0, _RPT)],
                        out_hbm.at[c, pl.ds(r0, _RPT)])

        @pl.when(s == _NS - 1)
        def _():
            pltpu.sync_copy(acc_sh.at[pl.ds(_RPT * _NS, _TAIL)],
                            out_hbm.at[c, pl.ds(_RPT * _NS, _TAIL)])

    return k(h, x_feat, src, dst, bases)


def _tc_preffn(x, W, b):
    def body(x_ref, w_ref, b_ref, o_ref):
        z = jnp.dot(x_ref[...], w_ref[...],
                    preferred_element_type=jnp.float32,
                    precision=lax.Precision.HIGHEST) + b_ref[...]
        o_ref[...] = _gelu(z)

    return pl.pallas_call(
        body,
        out_shape=jax.ShapeDtypeStruct((N, D), jnp.float32),
    )(x, W, b.reshape(1, D))


def _bn(z, g, b):
    mu = jnp.mean(z, axis=0, keepdims=True)
    var = jnp.mean((z - mu) ** 2, axis=0, keepdims=True)
    return (z - mu) / jnp.sqrt(var + 1e-5) * g + b


def _tc_ffn(x_feat, aggr, W1, b1, g1, be1, W2, b2, g2, be2):
    def body(xf, ag, w1, b1r, g1r, be1r, w2, b2r, g2r, be2r, o_ref):
        x = ag[0] + ag[1] - xf[...]
        y = jnp.dot(x, w1[...], preferred_element_type=jnp.float32,
                    precision=lax.Precision.HIGHEST) + b1r[...]
        y = _gelu(_bn(y, g1r[...], be1r[...]))
        y = jnp.dot(y, w2[...], preferred_element_type=jnp.float32,
                    precision=lax.Precision.HIGHEST) + b2r[...]
        y = _gelu(_bn(y, g2r[...], be2r[...]))
        o_ref[...] = x + y

    r = lambda v: v.reshape(1, D)
    return pl.pallas_call(
        body,
        out_shape=jax.ShapeDtypeStruct((N, D), jnp.float32),
    )(x_feat, aggr, W1, r(b1), r(g1), r(be1), W2, r(b2), r(g2), r(be2))


def kernel(x_feat, edge_index, bases, W_pre, b_pre, W1, b1, g1, be1, W2, b2, g2, be2):
    ei = edge_index.astype(jnp.int32)
    src = ei[0].reshape(_NW, _NCHUNK, 1, _CH)
    dst = ei[1].reshape(_NW, _NCHUNK, 1, _CH)
    h = _tc_preffn(x_feat, W_pre, b_pre)
    aggr = _sc_aggregate(h, x_feat, src, dst, bases)
    return _tc_ffn(x_feat, aggr, W1, b1, g1, be1, W2, b2, g2, be2)


# speedup vs baseline: 7.3561x; 1.0139x over previous
import functools

import jax
import jax.numpy as jnp
from jax import lax
from jax.experimental import pallas as pl
from jax.experimental.pallas import tpu as pltpu
from jax.experimental.pallas import tpu_sc as plsc

N = 10000
E = 320000
D = 128

_NC = 2
_NS = 16
_L = 16
_NW = _NC * _NS
_EPW = E // _NW
_CH = 40
_NCHUNK = _EPW // _CH
_RPT = 624
_TAIL = N - _RPT * _NS


def _gelu(z):
    return 0.5 * z * (1.0 + lax.erf(z * (2.0 ** -0.5)))


def _sc_aggregate(h, x_feat, src, dst, bases):
    mesh = plsc.VectorSubcoreMesh(core_axis_name="c", subcore_axis_name="s")

    @functools.partial(
        pl.kernel,
        mesh=mesh,
        out_type=jax.ShapeDtypeStruct((_NC, N, D), jnp.float32),
        scratch_types=[
            [pltpu.VMEM((_CH,), jnp.int32) for _ in range(6)],
            [pltpu.VMEM((_CH,), jnp.int32) for _ in range(6)],
            [pltpu.VMEM((_CH, D), jnp.float32) for _ in range(3)],
            [pltpu.VMEM((_CH, D), jnp.float32) for _ in range(3)],
            [pltpu.VMEM((_CH, D), jnp.float32) for _ in range(3)],
            pltpu.VMEM_SHARED((N, D), jnp.float32),
            [pltpu.SemaphoreType.DMA for _ in range(6)],
            [pltpu.SemaphoreType.DMA for _ in range(6)],
            [pltpu.SemaphoreType.DMA for _ in range(3)],
            [pltpu.SemaphoreType.DMA for _ in range(3)],
            [pltpu.SemaphoreType.DMA for _ in range(3)],
        ],
    )
    def k(h_hbm, x_hbm, src_hbm, dst_hbm, bases_hbm, out_hbm,
          sidx, didx, hv, bv, mv, acc_sh,
          sem_si, sem_di, sem_h, sem_b, sem_s):
        c = lax.axis_index("c")
        s = lax.axis_index("s")
        wid = s * _NC + c
        r0 = s * _RPT
        pltpu.sync_copy(x_hbm.at[pl.ds(r0, _RPT)], acc_sh.at[pl.ds(r0, _RPT)])

        @pl.when(s == _NS - 1)
        def _():
            pltpu.sync_copy(x_hbm.at[pl.ds(_RPT * _NS, _TAIL)],
                            acc_sh.at[pl.ds(_RPT * _NS, _TAIL)])

        plsc.subcore_barrier()

        ebase = wid * _EPW

        def start_sidx(i, q):
            pltpu.async_copy(src_hbm.at[pl.ds(ebase + i * _CH, _CH)], sidx[q], sem_si[q])

        def start_didx(i, q):
            pltpu.async_copy(dst_hbm.at[pl.ds(ebase + i * _CH, _CH)], didx[q], sem_di[q])

        def start_gather(i, b, q):
            pltpu.async_copy(h_hbm.at[sidx[q]], hv[b], sem_h[b])
            pltpu.async_copy(bases_hbm.at[pl.ds(ebase + i * _CH, _CH)],
                             bv[b], sem_b[b])

        for q in range(6):
            start_sidx(q, q)
        for q in range(3):
            start_didx(q, q)
        for b in range(3):
            pltpu.make_async_copy(src_hbm.at[pl.ds(ebase + b * _CH, _CH)], sidx[b], sem_si[b]).wait()
            start_gather(b, b, b)

        def chunk(i, b, q):
            q3 = (q + 3) % 6
            pltpu.make_async_copy(h_hbm.at[sidx[q]], hv[b],
                                  sem_h[b]).wait()
            pltpu.make_async_copy(
                bases_hbm.at[pl.ds(ebase + i * _CH, _CH)], bv[b],
                sem_b[b]).wait()

            @pl.when(i >= 3)
            def _():
                pltpu.make_async_copy(mv[b], acc_sh.at[didx[q]],
                                      sem_s[b]).wait()

            @pl.when(i + 3 < _NCHUNK)
            def _():
                start_didx(i + 3, q3)

            @plsc.parallel_loop(0, _CH, 1, unroll=8)
            def _mul(e):
                for j in range(D // _L):
                    sl = pl.ds(j * _L, _L)
                    mv[b][e, sl] = hv[b][e, sl] * bv[b][e, sl]

            pltpu.make_async_copy(dst_hbm.at[pl.ds(ebase + i * _CH, _CH)], didx[q], sem_di[q]).wait()
            pltpu.async_copy(mv[b], acc_sh.at[didx[q]], sem_s[b], add=True)

            @pl.when(i + 6 < _NCHUNK)
            def _():
                start_sidx(i + 6, q)

            @pl.when(i + 3 < _NCHUNK)
            def _():
                pltpu.make_async_copy(src_hbm.at[pl.ds(ebase + (i + 3) * _CH, _CH)], sidx[q3],
                                      sem_si[q3]).wait()
                start_gather(i + 3, b, q3)

        def six(i6, carry):
            for kk in range(6):
                chunk(i6 * 6 + kk, kk % 3, kk)
            return carry

        lax.fori_loop(0, _NCHUNK // 6, six, 0)
        for i in range(_NCHUNK - _NCHUNK % 6, _NCHUNK):
            chunk(i, i % 3, i % 6)
        for i in range(_NCHUNK - 3, _NCHUNK):
            pltpu.make_async_copy(
                mv[i % 3], acc_sh.at[didx[i % 6]],
                sem_s[i % 3]).wait()
        plsc.subcore_barrier()
        pltpu.sync_copy(acc_sh.at[pl.ds(r0, _RPT)],
                        out_hbm.at[c, pl.ds(r0, _RPT)])

        @pl.when(s == _NS - 1)
        def _():
            pltpu.sync_copy(acc_sh.at[pl.ds(_RPT * _NS, _TAIL)],
                            out_hbm.at[c, pl.ds(_RPT * _NS, _TAIL)])

    return k(h, x_feat, src, dst, bases)


def _tc_preffn(x, W, b):
    def body(x_ref, w_ref, b_ref, o_ref):
        z = jnp.dot(x_ref[...], w_ref[...],
                    preferred_element_type=jnp.float32,
                    precision=lax.Precision.HIGHEST) + b_ref[...]
        o_ref[...] = _gelu(z)

    return pl.pallas_call(
        body,
        out_shape=jax.ShapeDtypeStruct((N, D), jnp.float32),
    )(x, W, b.reshape(1, D))


def _bn(z, g, b):
    mu = jnp.mean(z, axis=0, keepdims=True)
    var = jnp.mean((z - mu) ** 2, axis=0, keepdims=True)
    return (z - mu) / jnp.sqrt(var + 1e-5) * g + b


def _tc_ffn(x_feat, aggr, W1, b1, g1, be1, W2, b2, g2, be2):
    def body(xf, ag, w1, b1r, g1r, be1r, w2, b2r, g2r, be2r, o_ref):
        x = ag[0] + ag[1] - xf[...]
        y = jnp.dot(x, w1[...], preferred_element_type=jnp.float32,
                    precision=lax.Precision.HIGHEST) + b1r[...]
        y = _gelu(_bn(y, g1r[...], be1r[...]))
        y = jnp.dot(y, w2[...], preferred_element_type=jnp.float32,
                    precision=lax.Precision.HIGHEST) + b2r[...]
        y = _gelu(_bn(y, g2r[...], be2r[...]))
        o_ref[...] = x + y

    r = lambda v: v.reshape(1, D)
    return pl.pallas_call(
        body,
        out_shape=jax.ShapeDtypeStruct((N, D), jnp.float32),
    )(x_feat, aggr, W1, r(b1), r(g1), r(be1), W2, r(b2), r(g2), r(be2))


def kernel(x_feat, edge_index, bases, W_pre, b_pre, W1, b1, g1, be1, W2, b2, g2, be2):
    ei = edge_index.astype(jnp.int32)
    src = ei[0]
    dst = ei[1]
    h = _tc_preffn(x_feat, W_pre, b_pre)
    aggr = _sc_aggregate(h, x_feat, src, dst, bases)
    return _tc_ffn(x_feat, aggr, W1, b1, g1, be1, W2, b2, g2, be2)


# speedup vs baseline: 7.9025x; 1.0743x over previous
import functools

import jax
import jax.numpy as jnp
from jax import lax
from jax.experimental import pallas as pl
from jax.experimental.pallas import tpu as pltpu
from jax.experimental.pallas import tpu_sc as plsc

N = 10000
E = 320000
D = 128

_NC = 2
_NS = 16
_L = 16
_NW = _NC * _NS
_EPW = E // _NW
_CH = 40
_NCHUNK = _EPW // _CH
_RPT = 624
_TAIL = N - _RPT * _NS


def _gelu(z):
    return 0.5 * z * (1.0 + lax.erf(z * (2.0 ** -0.5)))


def _sc_aggregate(h, x_feat, src, dst, bases):
    mesh = plsc.VectorSubcoreMesh(core_axis_name="c", subcore_axis_name="s")

    @functools.partial(
        pl.kernel,
        mesh=mesh,
        out_type=jax.ShapeDtypeStruct((_NC, N, D), jnp.float32),
        scratch_types=[
            [pltpu.VMEM((_CH,), jnp.int32) for _ in range(6)],
            [pltpu.VMEM((_CH,), jnp.int32) for _ in range(6)],
            [pltpu.VMEM((_CH, D), jnp.float32) for _ in range(3)],
            [pltpu.VMEM((_CH, D), jnp.float32) for _ in range(3)],
            [pltpu.VMEM((_CH, D), jnp.float32) for _ in range(3)],
            pltpu.VMEM_SHARED((N, D), jnp.float32),
            [pltpu.SemaphoreType.DMA for _ in range(6)],
            [pltpu.SemaphoreType.DMA for _ in range(6)],
            [pltpu.SemaphoreType.DMA for _ in range(3)],
            [pltpu.SemaphoreType.DMA for _ in range(3)],
            [pltpu.SemaphoreType.DMA for _ in range(3)],
        ],
    )
    def k(h_hbm, x_hbm, src_hbm, dst_hbm, bases_hbm, out_hbm,
          sidx, didx, hv, bv, mv, acc_sh,
          sem_si, sem_di, sem_h, sem_b, sem_s):
        c = lax.axis_index("c")
        s = lax.axis_index("s")
        wid = s * _NC + c
        r0 = s * _RPT
        pltpu.sync_copy(x_hbm.at[pl.ds(r0, _RPT)], acc_sh.at[pl.ds(r0, _RPT)])

        @pl.when(s == _NS - 1)
        def _():
            pltpu.sync_copy(x_hbm.at[pl.ds(_RPT * _NS, _TAIL)],
                            acc_sh.at[pl.ds(_RPT * _NS, _TAIL)])

        plsc.subcore_barrier()

        ebase = wid * _EPW

        def start_sidx(i, q):
            pltpu.async_copy(src_hbm.at[pl.ds(ebase + i * _CH, _CH)], sidx[q], sem_si[q])

        def start_didx(i, q):
            pltpu.async_copy(dst_hbm.at[pl.ds(ebase + i * _CH, _CH)], didx[q], sem_di[q])

        def start_gather(i, b, q):
            pltpu.async_copy(h_hbm.at[sidx[q]], hv[b], sem_h[b])
            pltpu.async_copy(bases_hbm.at[pl.ds(ebase + i * _CH, _CH)],
                             bv[b], sem_b[b])

        for q in range(6):
            start_sidx(q, q)
        for q in range(3):
            start_didx(q, q)
        for b in range(3):
            pltpu.make_async_copy(src_hbm.at[pl.ds(ebase + b * _CH, _CH)], sidx[b], sem_si[b]).wait()
            start_gather(b, b, b)

        def chunk(i, b, q):
            q3 = (q + 3) % 6
            pltpu.make_async_copy(h_hbm.at[sidx[q]], hv[b],
                                  sem_h[b]).wait()
            pltpu.make_async_copy(
                bases_hbm.at[pl.ds(ebase + i * _CH, _CH)], bv[b],
                sem_b[b]).wait()

            @pl.when(i >= 3)
            def _():
                pltpu.make_async_copy(mv[b], acc_sh.at[didx[q]],
                                      sem_s[b]).wait()

            @pl.when(i + 3 < _NCHUNK)
            def _():
                start_didx(i + 3, q3)

            @plsc.parallel_loop(0, _CH, 1, unroll=8)
            def _mul(e):
                for j in range(D // _L):
                    sl = pl.ds(j * _L, _L)
                    mv[b][e, sl] = hv[b][e, sl] * bv[b][e, sl]

            pltpu.make_async_copy(dst_hbm.at[pl.ds(ebase + i * _CH, _CH)], didx[q], sem_di[q]).wait()
            pltpu.async_copy(mv[b], acc_sh.at[didx[q]], sem_s[b], add=True)

            @pl.when(i + 6 < _NCHUNK)
            def _():
                start_sidx(i + 6, q)

            @pl.when(i + 3 < _NCHUNK)
            def _():
                pltpu.make_async_copy(src_hbm.at[pl.ds(ebase + (i + 3) * _CH, _CH)], sidx[q3],
                                      sem_si[q3]).wait()
                start_gather(i + 3, b, q3)

        def six(i6, carry):
            for kk in range(6):
                chunk(i6 * 6 + kk, kk % 3, kk)
            return carry

        lax.fori_loop(0, _NCHUNK // 6, six, 0)
        for i in range(_NCHUNK - _NCHUNK % 6, _NCHUNK):
            chunk(i, i % 3, i % 6)
        for i in range(_NCHUNK - 3, _NCHUNK):
            pltpu.make_async_copy(
                mv[i % 3], acc_sh.at[didx[i % 6]],
                sem_s[i % 3]).wait()
        plsc.subcore_barrier()
        pltpu.sync_copy(acc_sh.at[pl.ds(r0, _RPT)],
                        out_hbm.at[c, pl.ds(r0, _RPT)])

        @pl.when(s == _NS - 1)
        def _():
            pltpu.sync_copy(acc_sh.at[pl.ds(_RPT * _NS, _TAIL)],
                            out_hbm.at[c, pl.ds(_RPT * _NS, _TAIL)])

    return k(h, x_feat, src, dst, bases)


def _tc_preffn(x, W, b):
    def body(x_ref, w_ref, b_ref, o_ref):
        z = jnp.dot(x_ref[...], w_ref[...], preferred_element_type=jnp.float32) + b_ref[...]
        o_ref[...] = _gelu(z)

    return pl.pallas_call(
        body,
        out_shape=jax.ShapeDtypeStruct((N, D), jnp.float32),
    )(x, W, b.reshape(1, D))


def _bn(z, g, b):
    mu = jnp.mean(z, axis=0, keepdims=True)
    var = jnp.mean((z - mu) ** 2, axis=0, keepdims=True)
    return (z - mu) / jnp.sqrt(var + 1e-5) * g + b


def _tc_ffn(x_feat, aggr, W1, b1, g1, be1, W2, b2, g2, be2):
    def body(xf, ag, w1, b1r, g1r, be1r, w2, b2r, g2r, be2r, o_ref):
        x = ag[0] + ag[1] - xf[...]
        y = jnp.dot(x, w1[...], preferred_element_type=jnp.float32) + b1r[...]
        y = _gelu(_bn(y, g1r[...], be1r[...]))
        y = jnp.dot(y, w2[...], preferred_element_type=jnp.float32) + b2r[...]
        y = _gelu(_bn(y, g2r[...], be2r[...]))
        o_ref[...] = x + y

    r = lambda v: v.reshape(1, D)
    return pl.pallas_call(
        body,
        out_shape=jax.ShapeDtypeStruct((N, D), jnp.float32),
    )(x_feat, aggr, W1, r(b1), r(g1), r(be1), W2, r(b2), r(g2), r(be2))


def kernel(x_feat, edge_index, bases, W_pre, b_pre, W1, b1, g1, be1, W2, b2, g2, be2):
    ei = edge_index.astype(jnp.int32)
    src = ei[0]
    dst = ei[1]
    h = _tc_preffn(x_feat, W_pre, b_pre)
    aggr = _sc_aggregate(h, x_feat, src, dst, bases)
    return _tc_ffn(x_feat, aggr, W1, b1, g1, be1, W2, b2, g2, be2)
